# baseline (speedup 1.0000x reference)
import functools

import jax
import jax.numpy as jnp
from jax import lax
from jax.experimental import pallas as pl
from jax.experimental.pallas import tpu as pltpu
from jax.experimental.pallas import tpu_sc as plsc

N_NODES = 10000
N_EDGES = 160000
D_FEAT = 256
D_HALF = 128

NUM_CORES = 2
NUM_SUBCORES = 16
CHUNK = 128
E_PAD = 163840
ROWS_PT = E_PAD // NUM_SUBCORES // CHUNK
N_PAD = 10240
NODES_PT = N_PAD // NUM_SUBCORES
QROWS = 16
NQ = ROWS_PT // QROWS

_mesh = plsc.VectorSubcoreMesh(
    core_axis_name="c", subcore_axis_name="s",
    num_cores=NUM_CORES, num_subcores=NUM_SUBCORES)


@functools.partial(
    pl.kernel,
    out_type=(jax.ShapeDtypeStruct((N_PAD, D_HALF), jnp.float32),
              jax.ShapeDtypeStruct((N_PAD, D_HALF), jnp.float32)),
    mesh=_mesh,
    scratch_types=[
        pltpu.VMEM((ROWS_PT, CHUNK), jnp.int32),
        pltpu.VMEM((QROWS, CHUNK), jnp.int32),
        pltpu.VMEM((QROWS, CHUNK), jnp.float32),
        pltpu.VMEM((2, CHUNK, D_HALF), jnp.float32),
        pltpu.VMEM_SHARED((N_PAD, D_HALF), jnp.float32),
        pltpu.SemaphoreType.DMA,
        pltpu.SemaphoreType.DMA,
        pltpu.SemaphoreType.DMA,
        pltpu.SemaphoreType.DMA,
    ],
)
def _sc_agg(xlo, xhi, src2, dst2, ew2, zeros,
            out_lo, out_hi, sbuf, dbuf, wbuf, gbuf2, agg,
            gsem0, gsem1, ssem0, ssem1):
    cid = lax.axis_index("c")
    sid = lax.axis_index("s")

    pltpu.sync_copy(zeros.at[pl.ds(sid * NODES_PT, NODES_PT)],
                    agg.at[pl.ds(sid * NODES_PT, NODES_PT)])
    pltpu.sync_copy(src2.at[pl.ds(sid * ROWS_PT, ROWS_PT)], sbuf)
    plsc.subcore_barrier()

    def run(x_half):
        buf0 = gbuf2.at[0]
        buf1 = gbuf2.at[1]

        def scale(buf, k):
            def scale_body(g, c2):
                wv = wbuf[k, pl.ds(g * 16, 16)]
                for e in range(16):
                    row = g * 16 + e
                    w = jnp.full((16,), wv[e], jnp.float32)
                    for j in range(D_HALF // 16):
                        sl = pl.ds(j * 16, 16)
                        buf[row, sl] = buf[row, sl] * w
                return c2

            lax.fori_loop(0, CHUNK // 16, scale_body, 0)

        def gather_start(k, buf, sem):
            pltpu.async_copy(x_half.at[sbuf.at[k]], buf, sem)

        def gather_wait(buf, sem):
            pltpu.make_async_copy(x_half.at[sbuf.at[0]], buf, sem).wait()

        def scatter_start(buf, k, sem):
            pltpu.async_copy(buf, agg.at[dbuf.at[k]], sem, add=True)

        def scatter_wait(buf, sem):
            pltpu.make_async_copy(buf, agg.at[dbuf.at[0]], sem).wait()

        def quarter(q, carry):
            qbase = q * QROWS
            pltpu.sync_copy(dst2.at[pl.ds(sid * ROWS_PT + qbase, QROWS)], dbuf)
            pltpu.sync_copy(ew2.at[pl.ds(sid * ROWS_PT + qbase, QROWS)], wbuf)
            gather_start(qbase, buf0, gsem0)

            def body(kk, c2):
                q0 = kk * 2
                q1 = q0 + 1
                gather_wait(buf0, gsem0)

                @pl.when(kk > 0)
                def _():
                    scatter_wait(buf1, ssem1)

                gather_start(qbase + q1, buf1, gsem1)
                scale(buf0, q0)
                scatter_start(buf0, q0, ssem0)

                gather_wait(buf1, gsem1)
                scale(buf1, q1)
                scatter_wait(buf0, ssem0)

                @pl.when(kk < QROWS // 2 - 1)
                def _():
                    gather_start(qbase + q0 + 2, buf0, gsem0)

                scatter_start(buf1, q1, ssem1)
                return c2

            lax.fori_loop(0, QROWS // 2, body, 0)
            scatter_wait(buf1, ssem1)
            return carry

        lax.fori_loop(0, NQ, quarter, 0)

    @pl.when(cid == 0)
    def _():
        run(xlo)

    @pl.when(cid == 1)
    def _():
        run(xhi)

    plsc.subcore_barrier()

    @pl.when(cid == 0)
    def _():
        pltpu.sync_copy(agg.at[pl.ds(sid * NODES_PT, NODES_PT)],
                        out_lo.at[pl.ds(sid * NODES_PT, NODES_PT)])

    @pl.when(cid == 1)
    def _():
        pltpu.sync_copy(agg.at[pl.ds(sid * NODES_PT, NODES_PT)],
                        out_hi.at[pl.ds(sid * NODES_PT, NODES_PT)])


def _mm_body(alo_ref, ahi_ref, wlo_ref, whi_ref, b_ref, o_ref):
    acc = jnp.dot(alo_ref[...], wlo_ref[...], preferred_element_type=jnp.float32)
    acc = acc + jnp.dot(ahi_ref[...], whi_ref[...], preferred_element_type=jnp.float32)
    o_ref[...] = jnp.tanh(acc + b_ref[...])


_BN = 1024


def _mm(agg_lo, agg_hi, W, b):
    return pl.pallas_call(
        _mm_body,
        grid=(N_PAD // _BN,),
        in_specs=[
            pl.BlockSpec((_BN, D_HALF), lambda i: (i, 0)),
            pl.BlockSpec((_BN, D_HALF), lambda i: (i, 0)),
            pl.BlockSpec((D_HALF, D_FEAT), lambda i: (0, 0)),
            pl.BlockSpec((D_HALF, D_FEAT), lambda i: (0, 0)),
            pl.BlockSpec((1, D_FEAT), lambda i: (0, 0)),
        ],
        out_specs=pl.BlockSpec((_BN, D_FEAT), lambda i: (i, 0)),
        out_shape=jax.ShapeDtypeStruct((N_PAD, D_FEAT), jnp.float32),
    )(agg_lo, agg_hi, W[:D_HALF], W[D_HALF:], b.reshape(1, D_FEAT))


def kernel(x, edge_index, edge_weight, W, b):
    src = edge_index[0].astype(jnp.int32)
    dst = edge_index[1].astype(jnp.int32)
    ew = edge_weight.astype(jnp.float32)
    pad = E_PAD - N_EDGES
    src2 = jnp.concatenate([src, jnp.zeros((pad,), jnp.int32)]).reshape(-1, CHUNK)
    dst2 = jnp.concatenate([dst, jnp.zeros((pad,), jnp.int32)]).reshape(-1, CHUNK)
    ew2 = jnp.concatenate([ew, jnp.zeros((pad,), jnp.float32)]).reshape(-1, CHUNK)
    zeros = jnp.zeros((N_PAD, D_HALF), jnp.float32)
    x_lo = x[:, :D_HALF]
    x_hi = x[:, D_HALF:]
    agg_lo, agg_hi = _sc_agg(x_lo, x_hi, src2, dst2, ew2, zeros)
    return _mm(agg_lo, agg_hi, W, b)[:N_NODES]


# speedup vs baseline: 1.0553x; 1.0553x over previous
import functools

import jax
import jax.numpy as jnp
from jax import lax
from jax.experimental import pallas as pl
from jax.experimental.pallas import tpu as pltpu
from jax.experimental.pallas import tpu_sc as plsc

N_NODES = 10000
N_EDGES = 160000
D_FEAT = 256
D_HALF = 128

NUM_CORES = 2
NUM_SUBCORES = 16
CHUNK = 128
E_PAD = 163840
ROWS_PT = E_PAD // NUM_SUBCORES // CHUNK
N_PAD = 10240
NODES_PT = N_PAD // NUM_SUBCORES
QROWS = 16
NQ = ROWS_PT // QROWS

_mesh = plsc.VectorSubcoreMesh(
    core_axis_name="c", subcore_axis_name="s",
    num_cores=NUM_CORES, num_subcores=NUM_SUBCORES)


@functools.partial(
    pl.kernel,
    out_type=(jax.ShapeDtypeStruct((N_PAD, D_HALF), jnp.float32),
              jax.ShapeDtypeStruct((N_PAD, D_HALF), jnp.float32)),
    mesh=_mesh,
    scratch_types=[
        pltpu.VMEM((ROWS_PT, CHUNK), jnp.int32),
        pltpu.VMEM((QROWS, CHUNK), jnp.int32),
        pltpu.VMEM((QROWS, CHUNK), jnp.float32),
        pltpu.VMEM((2, CHUNK, D_HALF), jnp.float32),
        pltpu.VMEM_SHARED((N_PAD, D_HALF), jnp.float32),
        pltpu.SemaphoreType.DMA,
        pltpu.SemaphoreType.DMA,
        pltpu.SemaphoreType.DMA,
        pltpu.SemaphoreType.DMA,
    ],
)
def _sc_agg(xlo, xhi, src2, dst2, ew2, zeros,
            out_lo, out_hi, sbuf, dbuf, wbuf, gbuf2, agg,
            gsem0, gsem1, ssem0, ssem1):
    cid = lax.axis_index("c")
    sid = lax.axis_index("s")

    pltpu.sync_copy(zeros.at[pl.ds(sid * NODES_PT, NODES_PT)],
                    agg.at[pl.ds(sid * NODES_PT, NODES_PT)])
    pltpu.sync_copy(src2.at[pl.ds(sid * ROWS_PT, ROWS_PT)], sbuf)
    plsc.subcore_barrier()

    def run(x_half):
        buf0 = gbuf2.at[0]
        buf1 = gbuf2.at[1]

        def scale(buf, k):
            def scale_body(g, c2):
                wv = wbuf[k, pl.ds(g * 16, 16)]
                for e in range(16):
                    row = g * 16 + e
                    w = jnp.full((16,), wv[e], jnp.float32)
                    for j in range(D_HALF // 16):
                        sl = pl.ds(j * 16, 16)
                        buf[row, sl] = buf[row, sl] * w
                return c2

            lax.fori_loop(0, CHUNK // 16, scale_body, 0)

        def gather_start(k, buf, sem):
            pltpu.async_copy(x_half.at[sbuf.at[k]], buf, sem)

        def gather_wait(buf, sem):
            pltpu.make_async_copy(x_half.at[sbuf.at[0]], buf, sem).wait()

        def scatter_start(buf, k, sem):
            pltpu.async_copy(buf, agg.at[dbuf.at[k]], sem, add=True)

        def scatter_wait(buf, sem):
            pltpu.make_async_copy(buf, agg.at[dbuf.at[0]], sem).wait()

        def quarter(q, carry):
            qbase = q * QROWS
            gather_start(qbase, buf0, gsem0)
            pltpu.sync_copy(dst2.at[pl.ds(sid * ROWS_PT + qbase, QROWS)], dbuf)
            pltpu.sync_copy(ew2.at[pl.ds(sid * ROWS_PT + qbase, QROWS)], wbuf)

            def body(kk, c2):
                q0 = kk * 2
                q1 = q0 + 1
                gather_wait(buf0, gsem0)

                @pl.when(kk > 0)
                def _():
                    scatter_wait(buf1, ssem1)

                gather_start(qbase + q1, buf1, gsem1)
                scale(buf0, q0)
                scatter_start(buf0, q0, ssem0)

                gather_wait(buf1, gsem1)
                scatter_wait(buf0, ssem0)

                @pl.when(kk < QROWS // 2 - 1)
                def _():
                    gather_start(qbase + q0 + 2, buf0, gsem0)

                scale(buf1, q1)
                scatter_start(buf1, q1, ssem1)
                return c2

            lax.fori_loop(0, QROWS // 2, body, 0)
            scatter_wait(buf1, ssem1)
            return carry

        lax.fori_loop(0, NQ, quarter, 0)

    @pl.when(cid == 0)
    def _():
        run(xlo)

    @pl.when(cid == 1)
    def _():
        run(xhi)

    plsc.subcore_barrier()

    @pl.when(cid == 0)
    def _():
        pltpu.sync_copy(agg.at[pl.ds(sid * NODES_PT, NODES_PT)],
                        out_lo.at[pl.ds(sid * NODES_PT, NODES_PT)])

    @pl.when(cid == 1)
    def _():
        pltpu.sync_copy(agg.at[pl.ds(sid * NODES_PT, NODES_PT)],
                        out_hi.at[pl.ds(sid * NODES_PT, NODES_PT)])


def _mm_body(alo_ref, ahi_ref, wlo_ref, whi_ref, b_ref, o_ref):
    acc = jnp.dot(alo_ref[...], wlo_ref[...], preferred_element_type=jnp.float32)
    acc = acc + jnp.dot(ahi_ref[...], whi_ref[...], preferred_element_type=jnp.float32)
    o_ref[...] = jnp.tanh(acc + b_ref[...])


_BN = 1024


def _mm(agg_lo, agg_hi, W, b):
    return pl.pallas_call(
        _mm_body,
        grid=(N_PAD // _BN,),
        in_specs=[
            pl.BlockSpec((_BN, D_HALF), lambda i: (i, 0)),
            pl.BlockSpec((_BN, D_HALF), lambda i: (i, 0)),
            pl.BlockSpec((D_HALF, D_FEAT), lambda i: (0, 0)),
            pl.BlockSpec((D_HALF, D_FEAT), lambda i: (0, 0)),
            pl.BlockSpec((1, D_FEAT), lambda i: (0, 0)),
        ],
        out_specs=pl.BlockSpec((_BN, D_FEAT), lambda i: (i, 0)),
        out_shape=jax.ShapeDtypeStruct((N_PAD, D_FEAT), jnp.float32),
    )(agg_lo, agg_hi, W[:D_HALF], W[D_HALF:], b.reshape(1, D_FEAT))


def kernel(x, edge_index, edge_weight, W, b):
    src = edge_index[0].astype(jnp.int32)
    dst = edge_index[1].astype(jnp.int32)
    ew = edge_weight.astype(jnp.float32)
    pad = E_PAD - N_EDGES
    src2 = jnp.concatenate([src, jnp.zeros((pad,), jnp.int32)]).reshape(-1, CHUNK)
    dst2 = jnp.concatenate([dst, jnp.zeros((pad,), jnp.int32)]).reshape(-1, CHUNK)
    ew2 = jnp.concatenate([ew, jnp.zeros((pad,), jnp.float32)]).reshape(-1, CHUNK)
    zeros = jnp.zeros((N_PAD, D_HALF), jnp.float32)
    x_lo = x[:, :D_HALF]
    x_hi = x[:, D_HALF:]
    agg_lo, agg_hi = _sc_agg(x_lo, x_hi, src2, dst2, ew2, zeros)
    return _mm(agg_lo, agg_hi, W, b)[:N_NODES]
